# baseline (speedup 1.0000x reference)
import dataclasses
import functools

import jax
import jax.numpy as jnp
from jax import lax
from jax.experimental import pallas as pl
from jax.experimental.pallas import tpu as pltpu
from jax.experimental.pallas import tpu_sc as plsc

N = 10000
D = 128
E = 320000

NC = 2
NS = 16
NW = NC * NS
LPW = 320
NPAD = NW * LPW
EB = 6400
NBLK = E // EB
G = 64
NBUF = 4


def _seg_max_sc(t, src, dst):
    mesh = plsc.VectorSubcoreMesh(core_axis_name="c", subcore_axis_name="s")
    cp = pltpu.CompilerParams()
    if "needs_layout_passes" in pltpu.CompilerParams.__dataclass_fields__:
        cp = dataclasses.replace(cp, needs_layout_passes=False)

    @functools.partial(
        pl.kernel,
        out_type=jax.ShapeDtypeStruct((NPAD, D), jnp.float32),
        mesh=mesh,
        compiler_params=cp,
        scratch_types=[
            pltpu.VMEM((LPW + 1, D), jnp.float32),
            pltpu.VMEM((EB,), jnp.int32),
            pltpu.VMEM((EB,), jnp.int32),
            pltpu.VMEM((EB,), jnp.int32),
            pltpu.VMEM((EB,), jnp.int32),
            pltpu.VMEM((EB + G,), jnp.int32),
            pltpu.VMEM((EB + G,), jnp.int32),
            pltpu.VMEM((EB + G,), jnp.int32),
            pltpu.VMEM((EB + G,), jnp.int32),
            pltpu.VMEM((G, D), jnp.float32),
            pltpu.VMEM((G, D), jnp.float32),
            pltpu.VMEM((G, D), jnp.float32),
            pltpu.VMEM((G, D), jnp.float32),
            pltpu.SemaphoreType.DMA,
            pltpu.SemaphoreType.DMA,
            pltpu.SemaphoreType.DMA,
            pltpu.SemaphoreType.DMA,
            pltpu.SemaphoreType.DMA,
            pltpu.SemaphoreType.DMA,
            pltpu.SemaphoreType.DMA,
            pltpu.SemaphoreType.DMA,
        ],
    )
    def k(t_hbm, src_hbm, dst_hbm, out_hbm, agg, srcb0, srcb1, dstb0, dstb1,
          csrc0, csrc1, cdst0, cdst1, rows0, rows1, rows2, rows3,
          ss0, ss1, sd0, sd1, sg0, sg1, sg2, sg3):
        wid = lax.axis_index("s") * NC + lax.axis_index("c")
        lo = wid * LPW

        zero16 = jnp.zeros((16,), jnp.float32)
        izero16 = jnp.zeros((16,), jnp.int32)
        iota16 = lax.iota(jnp.int32, 16)
        junk16 = jnp.full((16,), LPW, jnp.int32)

        srcbs = (srcb0, srcb1)
        dstbs = (dstb0, dstb1)
        csrcs = (csrc0, csrc1)
        cdsts = (cdst0, cdst1)
        sss = (ss0, ss1)
        sds = (sd0, sd1)
        rowss = (rows0, rows1, rows2, rows3)
        sgs = (sg0, sg1, sg2, sg3)

        @pl.loop(0, LPW + 1)
        def _(r):
            for c in range(D // 16):
                agg[r, pl.ds(c * 16, 16)] = zero16

        @pl.loop(0, (EB + G) // 16)
        def _(i):
            sl = pl.ds(pl.multiple_of(i * 16, 16), 16)
            csrc0[sl] = izero16
            csrc1[sl] = izero16

        def fire_idx(b, w):
            eb0 = pl.multiple_of(b * EB, EB)
            pltpu.make_async_copy(src_hbm.at[pl.ds(eb0, EB)], srcbs[w],
                                  sss[w]).start()
            pltpu.make_async_copy(dst_hbm.at[pl.ds(eb0, EB)], dstbs[w],
                                  sds[w]).start()

        def wait_idx(b, w):
            eb0 = pl.multiple_of(b * EB, EB)
            pltpu.make_async_copy(src_hbm.at[pl.ds(eb0, EB)], srcbs[w],
                                  sss[w]).wait()
            pltpu.make_async_copy(dst_hbm.at[pl.ds(eb0, EB)], dstbs[w],
                                  sds[w]).wait()

        def fire_gather(w, g, i):
            base = pl.multiple_of(g * G, G)
            pltpu.make_async_copy(t_hbm.at[csrcs[w].at[pl.ds(base, G)]],
                                  rowss[i], sgs[i]).start()

        def wait_gather(w, g, i):
            base = pl.multiple_of(g * G, G)
            pltpu.make_async_copy(t_hbm.at[csrcs[w].at[pl.ds(base, G)]],
                                  rowss[i], sgs[i]).wait()

        def acc_block(w, g, i):
            rows = rowss[i]
            cdst = cdsts[w]
            base = pl.multiple_of(g * G, G)

            @pl.loop(0, G // 16)
            def _(q):
                qb = pl.multiple_of(q * 16, 16)
                d16 = cdst[pl.ds(base + qb, 16)]
                for l in range(16):
                    dloc = d16[l]
                    for c in range(D // 16):
                        slc = pl.ds(c * 16, 16)
                        agg[dloc, slc] = jnp.maximum(agg[dloc, slc],
                                                     rows[qb + l, slc])

        def filter_block(w):
            srcb, dstb = srcbs[w], dstbs[w]
            csrc, cdst = csrcs[w], cdsts[w]

            def chunk(i, cnt):
                sl = pl.ds(pl.multiple_of(i * 16, 16), 16)
                s16 = srcb[sl]
                dl = dstb[sl] - lo
                m = (dl >= 0) & (dl < LPW)
                mi = m.astype(jnp.int32)
                pos = lax.cumsum(mi) + (cnt - 1)
                plsc.store_scatter(csrc, [pos], s16, mask=m)
                plsc.store_scatter(cdst, [pos], dl, mask=m)
                return cnt + jnp.sum(mi)

            cnt = lax.fori_loop(0, EB // 16, chunk, 0)

            for q in range(G // 16):
                plsc.store_scatter(cdst, [cnt + q * 16 + iota16], junk16)

            return cnt

        def fire_first(w, cnt):
            ngb = (cnt + G - 1) // G
            for i in range(NBUF):
                @pl.when(i < ngb)
                def _():
                    fire_gather(w, i, i)

        def drain(w, cnt):
            ngb = (cnt + G - 1) // G

            def batch(t, _):
                base_g = t * NBUF
                for i in range(NBUF):
                    g = base_g + i

                    @pl.when(g < ngb)
                    def _():
                        wait_gather(w, g, i)
                        acc_block(w, g, i)

                        @pl.when(g + NBUF < ngb)
                        def _():
                            fire_gather(w, g + NBUF, i)

                return 0

            lax.fori_loop(0, (ngb + NBUF - 1) // NBUF, batch, 0)

        fire_idx(0, 0)
        fire_idx(1, 1)

        def outer(p, cnt1_prev):
            b0 = 2 * p
            wait_idx(b0, 0)
            cnt0 = filter_block(0)
            drain(1, cnt1_prev)

            @pl.when(b0 + 2 < NBLK)
            def _():
                fire_idx(b0 + 2, 0)

            fire_first(0, cnt0)
            wait_idx(b0 + 1, 1)
            cnt1 = filter_block(1)
            drain(0, cnt0)

            @pl.when(b0 + 3 < NBLK)
            def _():
                fire_idx(b0 + 3, 1)

            fire_first(1, cnt1)
            return cnt1

        cnt1_last = lax.fori_loop(0, NBLK // 2, outer, 0)
        drain(1, cnt1_last)

        pltpu.sync_copy(agg.at[pl.ds(0, LPW)], out_hbm.at[pl.ds(lo, LPW)])

    return k(t, src, dst)


def _dot(a, b):
    return jax.lax.dot_general(
        a, b, (((1,), (0,)), ((), ())),
        precision=jax.lax.Precision.HIGHEST,
        preferred_element_type=jnp.float32)


def _stage1(x, Wp0, bp0, Wf0_top):
    def body(x_ref, wp_ref, bp_ref, wft_ref, t_ref, p_ref):
        xv = x_ref[...]
        t_ref[...] = jnp.maximum(_dot(xv, wp_ref[...]) + bp_ref[...], 0.0)
        p_ref[...] = _dot(xv, wft_ref[...])

    return pl.pallas_call(
        body,
        out_shape=(jax.ShapeDtypeStruct((N, D), jnp.float32),
                   jax.ShapeDtypeStruct((N, D), jnp.float32)),
    )(x, Wp0, bp0, Wf0_top)


def _stage2(p0, agg0, Wf0_bot, bf0, gamma0, beta0, Wp1, bp1, Wf1_top):
    def body(p0_ref, agg_ref, wfb_ref, bf_ref, g_ref, b_ref, wp_ref, bp_ref,
             wft_ref, t_ref, p_ref):
        h = p0_ref[...] + _dot(agg_ref[...], wfb_ref[...]) + bf_ref[...]
        h = jnp.maximum(h, 0.0)
        mu = jnp.mean(h, axis=0, keepdims=True)
        dv = h - mu
        var = jnp.mean(dv * dv, axis=0, keepdims=True)
        hb = dv * lax.rsqrt(var + 1e-5) * g_ref[...] + b_ref[...]
        t_ref[...] = jnp.maximum(_dot(hb, wp_ref[...]) + bp_ref[...], 0.0)
        p_ref[...] = _dot(hb, wft_ref[...])

    return pl.pallas_call(
        body,
        out_shape=(jax.ShapeDtypeStruct((N, D), jnp.float32),
                   jax.ShapeDtypeStruct((N, D), jnp.float32)),
    )(p0, agg0, Wf0_bot, bf0, gamma0, beta0, Wp1, bp1, Wf1_top)


def _stage3(p1, agg1, Wf1_bot, bf1):
    def body(p1_ref, agg_ref, wfb_ref, bf_ref, o_ref):
        o_ref[...] = (p1_ref[...] + _dot(agg_ref[...], wfb_ref[...])
                      + bf_ref[...])

    return pl.pallas_call(
        body,
        out_shape=jax.ShapeDtypeStruct((N, D), jnp.float32),
    )(p1, agg1, Wf1_bot, bf1)


def kernel(x, edge_index, Wp0, bp0, Wf0, bf0, gamma0, beta0, Wp1, bp1, Wf1,
           bf1):
    src = edge_index[0].astype(jnp.int32)
    dst = edge_index[1].astype(jnp.int32)

    bp0r = bp0.reshape(1, D)
    bf0r = bf0.reshape(1, D)
    g0r = gamma0.reshape(1, D)
    b0r = beta0.reshape(1, D)
    bp1r = bp1.reshape(1, D)
    bf1r = bf1.reshape(1, D)

    t0, p0 = _stage1(x, Wp0, bp0r, Wf0[:D])
    agg0 = _seg_max_sc(t0, src, dst)[:N]
    t1, p1 = _stage2(p0, agg0, Wf0[D:], bf0r, g0r, b0r, Wp1, bp1r, Wf1[:D])
    agg1 = _seg_max_sc(t1, src, dst)[:N]
    return _stage3(p1, agg1, Wf1[D:], bf1r)


# speedup vs baseline: 1.7726x; 1.7726x over previous
import dataclasses
import functools

import jax
import jax.numpy as jnp
from jax import lax
from jax.experimental import pallas as pl
from jax.experimental.pallas import tpu as pltpu
from jax.experimental.pallas import tpu_sc as plsc

N = 10000
D = 128
E = 320000

NC = 2
NS = 16
NW = NC * NS
LPW = 320
NPAD = NW * LPW
EB = 4000
NBLK = E // EB
G = 64
NBUF = 4
CAP = 8192


def _seg_max_sc(t, src, dst):
    mesh = plsc.VectorSubcoreMesh(core_axis_name="c", subcore_axis_name="s")
    cp = pltpu.CompilerParams()
    if "needs_layout_passes" in pltpu.CompilerParams.__dataclass_fields__:
        cp = dataclasses.replace(cp, needs_layout_passes=False)

    @functools.partial(
        pl.kernel,
        out_type=jax.ShapeDtypeStruct((NPAD, D), jnp.float32),
        mesh=mesh,
        compiler_params=cp,
        scratch_types=[
            pltpu.VMEM((LPW + 1, D), jnp.float32),
            pltpu.VMEM((EB,), jnp.int32),
            pltpu.VMEM((EB,), jnp.int32),
            pltpu.VMEM((EB,), jnp.int32),
            pltpu.VMEM((EB,), jnp.int32),
            pltpu.VMEM((CAP,), jnp.int32),
            pltpu.VMEM((CAP,), jnp.int32),
            pltpu.VMEM((NBUF * G, D), jnp.float32),
            pltpu.SemaphoreType.DMA,
            pltpu.SemaphoreType.DMA,
            pltpu.SemaphoreType.DMA,
            pltpu.SemaphoreType.DMA,
            pltpu.SemaphoreType.DMA,
            pltpu.SemaphoreType.DMA,
            pltpu.SemaphoreType.DMA,
            pltpu.SemaphoreType.DMA,
        ],
    )
    def k(t_hbm, src_hbm, dst_hbm, out_hbm, agg, srcb0, srcb1, dstb0, dstb1,
          csrc, cdst, rowsb, ss0, ss1, sd0, sd1, sg0, sg1, sg2, sg3):
        wid = lax.axis_index("s") * NC + lax.axis_index("c")
        lo = wid * LPW

        zero16 = jnp.zeros((16,), jnp.float32)
        izero16 = jnp.zeros((16,), jnp.int32)
        iota16 = lax.iota(jnp.int32, 16)
        junk16 = jnp.full((16,), LPW, jnp.int32)

        srcbs = (srcb0, srcb1)
        dstbs = (dstb0, dstb1)
        sss = (ss0, ss1)
        sds = (sd0, sd1)
        sgs = (sg0, sg1, sg2, sg3)

        @pl.loop(0, LPW + 1)
        def _(r):
            for c in range(D // 16):
                agg[r, pl.ds(c * 16, 16)] = zero16

        @pl.loop(0, CAP // 16)
        def _(i):
            csrc[pl.ds(pl.multiple_of(i * 16, 16), 16)] = izero16

        def fire_idx(b, w):
            eb0 = pl.multiple_of(b * EB, EB)
            pltpu.make_async_copy(src_hbm.at[pl.ds(eb0, EB)], srcbs[w],
                                  sss[w]).start()
            pltpu.make_async_copy(dst_hbm.at[pl.ds(eb0, EB)], dstbs[w],
                                  sds[w]).start()

        def wait_idx(b, w):
            eb0 = pl.multiple_of(b * EB, EB)
            pltpu.make_async_copy(src_hbm.at[pl.ds(eb0, EB)], srcbs[w],
                                  sss[w]).wait()
            pltpu.make_async_copy(dst_hbm.at[pl.ds(eb0, EB)], dstbs[w],
                                  sds[w]).wait()

        def gather_copy(g, i):
            rbase = pl.multiple_of((g * G) & (CAP - 1), G)
            sbase = pl.multiple_of(i * G, G)
            return pltpu.make_async_copy(
                t_hbm.at[csrc.at[pl.ds(rbase, G)]],
                rowsb.at[pl.ds(sbase, G)], sgs[i])

        def fire_gather(g):
            for i in range(NBUF):
                @pl.when(g & (NBUF - 1) == i)
                def _():
                    gather_copy(g, i).start()

        def wait_gather(g):
            for i in range(NBUF):
                @pl.when(g & (NBUF - 1) == i)
                def _():
                    gather_copy(g, i).wait()

        def acc_granule(g):
            rbase = pl.multiple_of((g * G) & (CAP - 1), G)
            slot = (g & (NBUF - 1)) * G
            for q in range(G // 16):
                d16 = cdst[pl.ds(rbase + pl.multiple_of(q * 16, 16), 16)]
                for l in range(16):
                    dloc = d16[l]
                    j = slot + q * 16 + l
                    for c in range(D // 16):
                        slc = pl.ds(c * 16, 16)
                        agg[dloc, slc] = jnp.maximum(agg[dloc, slc],
                                                     rowsb[j, slc])

        def filter_block(w, fill):
            srcb, dstb = srcbs[w], dstbs[w]

            def chunk(i, f):
                sl = pl.ds(pl.multiple_of(i * 16, 16), 16)
                s16 = srcb[sl]
                dl = dstb[sl] - lo
                m = (dl >= 0) & (dl < LPW)
                mi = m.astype(jnp.int32)
                pos = (lax.cumsum(mi) + (f - 1)) & (CAP - 1)
                plsc.store_scatter(csrc, [pos], s16, mask=m)
                plsc.store_scatter(cdst, [pos], dl, mask=m)
                return f + jnp.sum(mi)

            return lax.fori_loop(0, EB // 16, chunk, fill)

        def pump(navail, fill, gd, gf):
            def cond(s):
                gd_, gf_ = s
                can_fire = (gf_ < navail) & (gf_ < gd_ + NBUF)
                need_room = (fill - gd_ * G) > (CAP - EB)
                return can_fire | need_room

            def body(s):
                gd_, gf_ = s
                can_fire = (gf_ < navail) & (gf_ < gd_ + NBUF)

                @pl.when(can_fire)
                def _():
                    fire_gather(gf_)

                @pl.when(jnp.logical_not(can_fire))
                def _():
                    wait_gather(gd_)
                    acc_granule(gd_)

                return (gd_ + jnp.where(can_fire, 0, 1),
                        gf_ + jnp.where(can_fire, 1, 0))

            return lax.while_loop(cond, body, (gd, gf))

        fire_idx(0, 0)
        fire_idx(1, 1)

        def outer(p, state):
            fill, gd, gf = state
            b0 = 2 * p
            for w in range(2):
                b = b0 + w
                wait_idx(b, w)
                fill = filter_block(w, fill)

                @pl.when(b + 2 < NBLK)
                def _():
                    fire_idx(b + 2, w)

                gd, gf = pump(fill // G, fill, gd, gf)
            return fill, gd, gf

        fill, gd, gf = lax.fori_loop(0, NBLK // 2, outer, (0, 0, 0))

        for q in range(G // 16):
            pos = (fill + q * 16 + iota16) & (CAP - 1)
            plsc.store_scatter(cdst, [pos], junk16)
        nfin = (fill + G - 1) // G

        def fincond(s):
            gd_, gf_ = s
            return gd_ < nfin

        def finbody(s):
            gd_, gf_ = s
            can_fire = (gf_ < nfin) & (gf_ < gd_ + NBUF)

            @pl.when(can_fire)
            def _():
                fire_gather(gf_)

            @pl.when(jnp.logical_not(can_fire))
            def _():
                wait_gather(gd_)
                acc_granule(gd_)

            return (gd_ + jnp.where(can_fire, 0, 1),
                    gf_ + jnp.where(can_fire, 1, 0))

        lax.while_loop(fincond, finbody, (gd, gf))

        pltpu.sync_copy(agg.at[pl.ds(0, LPW)], out_hbm.at[pl.ds(lo, LPW)])

    return k(t, src, dst)


def _dot(a, b):
    return jax.lax.dot_general(
        a, b, (((1,), (0,)), ((), ())),
        precision=jax.lax.Precision.HIGHEST,
        preferred_element_type=jnp.float32)


def _stage1(x, Wp0, bp0, Wf0_top):
    def body(x_ref, wp_ref, bp_ref, wft_ref, t_ref, p_ref):
        xv = x_ref[...]
        t_ref[...] = jnp.maximum(_dot(xv, wp_ref[...]) + bp_ref[...], 0.0)
        p_ref[...] = _dot(xv, wft_ref[...])

    return pl.pallas_call(
        body,
        out_shape=(jax.ShapeDtypeStruct((N, D), jnp.float32),
                   jax.ShapeDtypeStruct((N, D), jnp.float32)),
    )(x, Wp0, bp0, Wf0_top)


def _stage2(p0, agg0, Wf0_bot, bf0, gamma0, beta0, Wp1, bp1, Wf1_top):
    def body(p0_ref, agg_ref, wfb_ref, bf_ref, g_ref, b_ref, wp_ref, bp_ref,
             wft_ref, t_ref, p_ref):
        h = p0_ref[...] + _dot(agg_ref[...], wfb_ref[...]) + bf_ref[...]
        h = jnp.maximum(h, 0.0)
        mu = jnp.mean(h, axis=0, keepdims=True)
        dv = h - mu
        var = jnp.mean(dv * dv, axis=0, keepdims=True)
        hb = dv * lax.rsqrt(var + 1e-5) * g_ref[...] + b_ref[...]
        t_ref[...] = jnp.maximum(_dot(hb, wp_ref[...]) + bp_ref[...], 0.0)
        p_ref[...] = _dot(hb, wft_ref[...])

    return pl.pallas_call(
        body,
        out_shape=(jax.ShapeDtypeStruct((N, D), jnp.float32),
                   jax.ShapeDtypeStruct((N, D), jnp.float32)),
    )(p0, agg0, Wf0_bot, bf0, gamma0, beta0, Wp1, bp1, Wf1_top)


def _stage3(p1, agg1, Wf1_bot, bf1):
    def body(p1_ref, agg_ref, wfb_ref, bf_ref, o_ref):
        o_ref[...] = (p1_ref[...] + _dot(agg_ref[...], wfb_ref[...])
                      + bf_ref[...])

    return pl.pallas_call(
        body,
        out_shape=jax.ShapeDtypeStruct((N, D), jnp.float32),
    )(p1, agg1, Wf1_bot, bf1)


def kernel(x, edge_index, Wp0, bp0, Wf0, bf0, gamma0, beta0, Wp1, bp1, Wf1,
           bf1):
    src = edge_index[0].astype(jnp.int32)
    dst = edge_index[1].astype(jnp.int32)

    bp0r = bp0.reshape(1, D)
    bf0r = bf0.reshape(1, D)
    g0r = gamma0.reshape(1, D)
    b0r = beta0.reshape(1, D)
    bp1r = bp1.reshape(1, D)
    bf1r = bf1.reshape(1, D)

    t0, p0 = _stage1(x, Wp0, bp0r, Wf0[:D])
    agg0 = _seg_max_sc(t0, src, dst)[:N]
    t1, p1 = _stage2(p0, agg0, Wf0[D:], bf0r, g0r, b0r, Wp1, bp1r, Wf1[:D])
    agg1 = _seg_max_sc(t1, src, dst)[:N]
    return _stage3(p1, agg1, Wf1[D:], bf1r)


# speedup vs baseline: 2.1810x; 1.2304x over previous
import dataclasses
import functools

import jax
import jax.numpy as jnp
from jax import lax
from jax.experimental import pallas as pl
from jax.experimental.pallas import tpu as pltpu
from jax.experimental.pallas import tpu_sc as plsc

N = 10000
D = 128
E = 320000

NC = 2
NS = 16
NW = NC * NS
LPW = 320
NPAD = NW * LPW
EB = 4000
NBLK = E // EB
G = 64
NBUF = 4
CAP = 8192


def _seg_max_sc(t, src, dst):
    mesh = plsc.VectorSubcoreMesh(core_axis_name="c", subcore_axis_name="s")
    cp = pltpu.CompilerParams()
    if "needs_layout_passes" in pltpu.CompilerParams.__dataclass_fields__:
        cp = dataclasses.replace(cp, needs_layout_passes=False)

    @functools.partial(
        pl.kernel,
        out_type=jax.ShapeDtypeStruct((NPAD, D), jnp.float32),
        mesh=mesh,
        compiler_params=cp,
        scratch_types=[
            pltpu.VMEM((LPW + 1, D), jnp.float32),
            pltpu.VMEM((EB,), jnp.int32),
            pltpu.VMEM((EB,), jnp.int32),
            pltpu.VMEM((EB,), jnp.int32),
            pltpu.VMEM((EB,), jnp.int32),
            pltpu.VMEM((CAP,), jnp.int32),
            pltpu.VMEM((CAP,), jnp.int32),
            pltpu.VMEM((NBUF * G, D), jnp.float32),
            pltpu.SemaphoreType.DMA,
            pltpu.SemaphoreType.DMA,
            pltpu.SemaphoreType.DMA,
            pltpu.SemaphoreType.DMA,
            pltpu.SemaphoreType.DMA,
            pltpu.SemaphoreType.DMA,
            pltpu.SemaphoreType.DMA,
            pltpu.SemaphoreType.DMA,
        ],
    )
    def k(t_hbm, src_hbm, dst_hbm, out_hbm, agg, srcb0, srcb1, dstb0, dstb1,
          csrc, cdst, rowsb, ss0, ss1, sd0, sd1, sg0, sg1, sg2, sg3):
        wid = lax.axis_index("s") * NC + lax.axis_index("c")
        lo = wid * LPW

        zero16 = jnp.zeros((16,), jnp.float32)
        izero16 = jnp.zeros((16,), jnp.int32)
        iota16 = lax.iota(jnp.int32, 16)
        junk16 = jnp.full((16,), LPW, jnp.int32)

        srcbs = (srcb0, srcb1)
        dstbs = (dstb0, dstb1)
        sss = (ss0, ss1)
        sds = (sd0, sd1)
        sgs = (sg0, sg1, sg2, sg3)

        @pl.loop(0, LPW + 1)
        def _(r):
            for c in range(D // 16):
                agg[r, pl.ds(c * 16, 16)] = zero16

        @pl.loop(0, CAP // 16)
        def _(i):
            csrc[pl.ds(pl.multiple_of(i * 16, 16), 16)] = izero16

        def fire_idx(b, w):
            eb0 = pl.multiple_of(b * EB, EB)
            pltpu.make_async_copy(src_hbm.at[pl.ds(eb0, EB)], srcbs[w],
                                  sss[w]).start()
            pltpu.make_async_copy(dst_hbm.at[pl.ds(eb0, EB)], dstbs[w],
                                  sds[w]).start()

        def wait_idx(b, w):
            eb0 = pl.multiple_of(b * EB, EB)
            pltpu.make_async_copy(src_hbm.at[pl.ds(eb0, EB)], srcbs[w],
                                  sss[w]).wait()
            pltpu.make_async_copy(dst_hbm.at[pl.ds(eb0, EB)], dstbs[w],
                                  sds[w]).wait()

        def gather_copy(g, i):
            rbase = pl.multiple_of((g * G) & (CAP - 1), G)
            sbase = pl.multiple_of(i * G, G)
            return pltpu.make_async_copy(
                t_hbm.at[csrc.at[pl.ds(rbase, G)]],
                rowsb.at[pl.ds(sbase, G)], sgs[i])

        def fire_gather(g):
            for i in range(NBUF):
                @pl.when(g & (NBUF - 1) == i)
                def _():
                    gather_copy(g, i).start()

        def wait_gather(g):
            for i in range(NBUF):
                @pl.when(g & (NBUF - 1) == i)
                def _():
                    gather_copy(g, i).wait()

        def acc_granule(g):
            rbase = pl.multiple_of((g * G) & (CAP - 1), G)
            slot = (g & (NBUF - 1)) * G
            for q in range(G // 16):
                d16 = cdst[pl.ds(rbase + pl.multiple_of(q * 16, 16), 16)]
                for l in range(16):
                    dloc = d16[l]
                    j = slot + q * 16 + l
                    for c in range(D // 16):
                        slc = pl.ds(c * 16, 16)
                        agg[dloc, slc] = jnp.maximum(agg[dloc, slc],
                                                     rowsb[j, slc])

        def filter_block(w, fill):
            srcb, dstb = srcbs[w], dstbs[w]

            def chunk(i, f):
                sl = pl.ds(pl.multiple_of(i * 16, 16), 16)
                s16 = srcb[sl]
                dl = dstb[sl] - lo
                m = (dl >= 0) & (dl < LPW)
                mi = m.astype(jnp.int32)
                pos = (lax.cumsum(mi) + (f - 1)) & (CAP - 1)
                plsc.store_scatter(csrc, [pos], s16, mask=m)
                plsc.store_scatter(cdst, [pos], dl, mask=m)
                return f + jnp.sum(mi)

            return plsc.parallel_loop(0, EB // 16, unroll=4,
                                      carry=fill)(chunk)

        def pump(navail, fill, gd, gf):
            def cond(s):
                gd_, gf_ = s
                can_fire = (gf_ < navail) & (gf_ < gd_ + NBUF)
                need_room = (fill - gd_ * G) > (CAP - EB)
                return can_fire | need_room

            def body(s):
                gd_, gf_ = s
                can_fire = (gf_ < navail) & (gf_ < gd_ + NBUF)

                @pl.when(can_fire)
                def _():
                    fire_gather(gf_)

                @pl.when(jnp.logical_not(can_fire))
                def _():
                    wait_gather(gd_)
                    acc_granule(gd_)

                return (gd_ + jnp.where(can_fire, 0, 1),
                        gf_ + jnp.where(can_fire, 1, 0))

            return lax.while_loop(cond, body, (gd, gf))

        fire_idx(0, 0)
        fire_idx(1, 1)

        def outer(p, state):
            fill, gd, gf = state
            b0 = 2 * p
            for w in range(2):
                b = b0 + w
                wait_idx(b, w)
                fill = filter_block(w, fill)

                @pl.when(b + 2 < NBLK)
                def _():
                    fire_idx(b + 2, w)

                gd, gf = pump(fill // G, fill, gd, gf)
            return fill, gd, gf

        fill, gd, gf = lax.fori_loop(0, NBLK // 2, outer, (0, 0, 0))

        for q in range(G // 16):
            pos = (fill + q * 16 + iota16) & (CAP - 1)
            plsc.store_scatter(cdst, [pos], junk16)
        nfin = (fill + G - 1) // G

        def fincond(s):
            gd_, gf_ = s
            return gd_ < nfin

        def finbody(s):
            gd_, gf_ = s
            can_fire = (gf_ < nfin) & (gf_ < gd_ + NBUF)

            @pl.when(can_fire)
            def _():
                fire_gather(gf_)

            @pl.when(jnp.logical_not(can_fire))
            def _():
                wait_gather(gd_)
                acc_granule(gd_)

            return (gd_ + jnp.where(can_fire, 0, 1),
                    gf_ + jnp.where(can_fire, 1, 0))

        lax.while_loop(fincond, finbody, (gd, gf))

        pltpu.sync_copy(agg.at[pl.ds(0, LPW)], out_hbm.at[pl.ds(lo, LPW)])

    return k(t, src, dst)


def _dot(a, b):
    return jax.lax.dot_general(
        a, b, (((1,), (0,)), ((), ())),
        precision=jax.lax.Precision.HIGHEST,
        preferred_element_type=jnp.float32)


def _stage1(x, Wp0, bp0, Wf0_top):
    def body(x_ref, wp_ref, bp_ref, wft_ref, t_ref, p_ref):
        xv = x_ref[...]
        t_ref[...] = jnp.maximum(_dot(xv, wp_ref[...]) + bp_ref[...], 0.0)
        p_ref[...] = _dot(xv, wft_ref[...])

    return pl.pallas_call(
        body,
        out_shape=(jax.ShapeDtypeStruct((N, D), jnp.float32),
                   jax.ShapeDtypeStruct((N, D), jnp.float32)),
    )(x, Wp0, bp0, Wf0_top)


def _stage2(p0, agg0, Wf0_bot, bf0, gamma0, beta0, Wp1, bp1, Wf1_top):
    def body(p0_ref, agg_ref, wfb_ref, bf_ref, g_ref, b_ref, wp_ref, bp_ref,
             wft_ref, t_ref, p_ref):
        h = p0_ref[...] + _dot(agg_ref[...], wfb_ref[...]) + bf_ref[...]
        h = jnp.maximum(h, 0.0)
        mu = jnp.mean(h, axis=0, keepdims=True)
        dv = h - mu
        var = jnp.mean(dv * dv, axis=0, keepdims=True)
        hb = dv * lax.rsqrt(var + 1e-5) * g_ref[...] + b_ref[...]
        t_ref[...] = jnp.maximum(_dot(hb, wp_ref[...]) + bp_ref[...], 0.0)
        p_ref[...] = _dot(hb, wft_ref[...])

    return pl.pallas_call(
        body,
        out_shape=(jax.ShapeDtypeStruct((N, D), jnp.float32),
                   jax.ShapeDtypeStruct((N, D), jnp.float32)),
    )(p0, agg0, Wf0_bot, bf0, gamma0, beta0, Wp1, bp1, Wf1_top)


def _stage3(p1, agg1, Wf1_bot, bf1):
    def body(p1_ref, agg_ref, wfb_ref, bf_ref, o_ref):
        o_ref[...] = (p1_ref[...] + _dot(agg_ref[...], wfb_ref[...])
                      + bf_ref[...])

    return pl.pallas_call(
        body,
        out_shape=jax.ShapeDtypeStruct((N, D), jnp.float32),
    )(p1, agg1, Wf1_bot, bf1)


def kernel(x, edge_index, Wp0, bp0, Wf0, bf0, gamma0, beta0, Wp1, bp1, Wf1,
           bf1):
    src = edge_index[0].astype(jnp.int32)
    dst = edge_index[1].astype(jnp.int32)

    bp0r = bp0.reshape(1, D)
    bf0r = bf0.reshape(1, D)
    g0r = gamma0.reshape(1, D)
    b0r = beta0.reshape(1, D)
    bp1r = bp1.reshape(1, D)
    bf1r = bf1.reshape(1, D)

    t0, p0 = _stage1(x, Wp0, bp0r, Wf0[:D])
    agg0 = _seg_max_sc(t0, src, dst)[:N]
    t1, p1 = _stage2(p0, agg0, Wf0[D:], bf0r, g0r, b0r, Wp1, bp1r, Wf1[:D])
    agg1 = _seg_max_sc(t1, src, dst)[:N]
    return _stage3(p1, agg1, Wf1[D:], bf1r)
